# Initial kernel scaffold; baseline (speedup 1.0000x reference)
import jax
import jax.numpy as jnp
from jax.experimental import pallas as pl


def kernel(x, w_gate, b_gate):
    raise NotImplementedError("write your pallas kernel here")


# speedup vs baseline: 3.7557x; 3.7557x over previous
import functools

import jax
import jax.numpy as jnp
from jax.experimental import pallas as pl
from jax.experimental.pallas import tpu as pltpu

_DIM = 768
_NE = 64
_EPS = 1e-6
_TILE = 512


def _pass1_body(x_ref, w_ref, b_ref, v_ref, e_ref, d_ref):
    i = pl.program_id(0)
    logits = jax.lax.dot_general(
        x_ref[...], w_ref[...], (((1,), (1,)), ((), ())),
        preferred_element_type=jnp.float32)
    logits = logits + b_ref[...]
    m = jnp.max(logits, axis=1, keepdims=True)
    s = jnp.sum(jnp.exp(logits - m), axis=1, keepdims=True)
    v = 1.0 / s[:, 0]
    e = jnp.argmax(logits, axis=1).astype(jnp.int32)
    v_ref[0, 0, :] = v
    e_ref[0, 0, :] = e
    oh = jax.lax.broadcasted_iota(jnp.int32, (_TILE, _NE), 1) == e[:, None]
    contrib = jnp.sum(jnp.where(oh, v[:, None], 0.0), axis=0)

    @pl.when(i == 0)
    def _():
        d_ref[...] = jnp.zeros_like(d_ref)

    d_ref[0, :] += contrib


def _pass2_body(v_ref, e_ref, d_ref, o_ref, *, capacity):
    recip = capacity / (d_ref[0, :] + _EPS)
    v = v_ref[0, 0, :]
    e = e_ref[0, 0, :]
    oh = jax.lax.broadcasted_iota(jnp.int32, (_TILE, _NE), 1) == e[:, None]
    o_ref[...] = jnp.where(oh, v[:, None] * recip[None, :], 0.0)


def kernel(x, w_gate, b_gate):
    n, dim = x.shape
    ne = w_gate.shape[0]
    capacity = float(n)
    num_tiles = n // _TILE
    b2 = b_gate.reshape(1, ne)

    v3, e3, d = pl.pallas_call(
        _pass1_body,
        grid=(num_tiles,),
        in_specs=[
            pl.BlockSpec((_TILE, dim), lambda i: (i, 0)),
            pl.BlockSpec((ne, dim), lambda i: (0, 0)),
            pl.BlockSpec((1, ne), lambda i: (0, 0)),
        ],
        out_specs=[
            pl.BlockSpec((1, 1, _TILE), lambda i: (i, 0, 0)),
            pl.BlockSpec((1, 1, _TILE), lambda i: (i, 0, 0)),
            pl.BlockSpec((1, ne), lambda i: (0, 0)),
        ],
        out_shape=[
            jax.ShapeDtypeStruct((num_tiles, 1, _TILE), jnp.float32),
            jax.ShapeDtypeStruct((num_tiles, 1, _TILE), jnp.int32),
            jax.ShapeDtypeStruct((1, ne), jnp.float32),
        ],
        compiler_params=pltpu.CompilerParams(
            dimension_semantics=("arbitrary",)),
    )(x, w_gate, b2)

    out = pl.pallas_call(
        functools.partial(_pass2_body, capacity=capacity),
        grid=(num_tiles,),
        in_specs=[
            pl.BlockSpec((1, 1, _TILE), lambda i: (i, 0, 0)),
            pl.BlockSpec((1, 1, _TILE), lambda i: (i, 0, 0)),
            pl.BlockSpec((1, ne), lambda i: (0, 0)),
        ],
        out_specs=pl.BlockSpec((_TILE, ne), lambda i: (i, 0)),
        out_shape=jax.ShapeDtypeStruct((n, ne), jnp.float32),
        compiler_params=pltpu.CompilerParams(
            dimension_semantics=("arbitrary",)),
    )(v3, e3, d)
    return out


# speedup vs baseline: 6.9090x; 1.8396x over previous
import functools

import jax
import jax.numpy as jnp
from jax.experimental import pallas as pl
from jax.experimental.pallas import tpu as pltpu

_NE = 64
_EPS = 1e-6
_TILE = 1024


def _pass1_body(x_ref, w_ref, b_ref, v_ref, e_ref, dacc_ref):
    i = pl.program_id(0)
    lt = jax.lax.dot_general(
        w_ref[...], x_ref[...], (((1,), (1,)), ((), ())),
        preferred_element_type=jnp.float32)
    lt = lt + b_ref[...]
    m = jnp.max(lt, axis=0, keepdims=True)
    s = jnp.sum(jnp.exp(lt - m), axis=0, keepdims=True)
    v = 1.0 / s
    iota = jax.lax.broadcasted_iota(jnp.int32, (_NE, _TILE), 0)
    e = jnp.min(jnp.where(lt == m, iota, _NE), axis=0, keepdims=True)
    v_ref[0, 0, :] = v[0]
    e_ref[0, 0, :] = e[0]
    contrib = jnp.where(iota == e, v, 0.0)

    @pl.when(i == 0)
    def _():
        dacc_ref[...] = jnp.zeros_like(dacc_ref)

    dacc_ref[...] += contrib


def _pass2_body(v_ref, e_ref, dacc_ref, o_ref, *, capacity):
    denom = jnp.sum(dacc_ref[...], axis=1, keepdims=True) + _EPS
    recip = capacity / denom
    v = v_ref[0, 0, :][None, :]
    e = e_ref[0, 0, :][None, :]
    iota = jax.lax.broadcasted_iota(jnp.int32, (_NE, _TILE), 0)
    out_t = jnp.where(iota == e, v * recip, 0.0)
    o_ref[...] = out_t.T


def kernel(x, w_gate, b_gate):
    n, dim = x.shape
    ne = w_gate.shape[0]
    capacity = float(n)
    num_tiles = n // _TILE
    b2 = b_gate.reshape(ne, 1)

    v3, e3, dacc = pl.pallas_call(
        _pass1_body,
        grid=(num_tiles,),
        in_specs=[
            pl.BlockSpec((_TILE, dim), lambda i: (i, 0)),
            pl.BlockSpec((ne, dim), lambda i: (0, 0)),
            pl.BlockSpec((ne, 1), lambda i: (0, 0)),
        ],
        out_specs=[
            pl.BlockSpec((1, 1, _TILE), lambda i: (i, 0, 0)),
            pl.BlockSpec((1, 1, _TILE), lambda i: (i, 0, 0)),
            pl.BlockSpec((ne, _TILE), lambda i: (0, 0)),
        ],
        out_shape=[
            jax.ShapeDtypeStruct((num_tiles, 1, _TILE), jnp.float32),
            jax.ShapeDtypeStruct((num_tiles, 1, _TILE), jnp.int32),
            jax.ShapeDtypeStruct((ne, _TILE), jnp.float32),
        ],
        compiler_params=pltpu.CompilerParams(
            dimension_semantics=("arbitrary",)),
    )(x, w_gate, b2)

    out = pl.pallas_call(
        functools.partial(_pass2_body, capacity=capacity),
        grid=(num_tiles,),
        in_specs=[
            pl.BlockSpec((1, 1, _TILE), lambda i: (i, 0, 0)),
            pl.BlockSpec((1, 1, _TILE), lambda i: (i, 0, 0)),
            pl.BlockSpec((ne, _TILE), lambda i: (0, 0)),
        ],
        out_specs=pl.BlockSpec((_TILE, ne), lambda i: (i, 0)),
        out_shape=jax.ShapeDtypeStruct((n, ne), jnp.float32),
        compiler_params=pltpu.CompilerParams(
            dimension_semantics=("arbitrary",)),
    )(v3, e3, dacc)
    return out


# speedup vs baseline: 8.6389x; 1.2504x over previous
import functools

import jax
import jax.numpy as jnp
from jax.experimental import pallas as pl
from jax.experimental.pallas import tpu as pltpu

_NE = 64
_EPS = 1e-6
_TILE = 2048


def _pass1_body(x_ref, w_ref, b_ref, v_ref, e_ref, dacc_ref):
    i = pl.program_id(0)
    lt = jax.lax.dot_general(
        w_ref[...], x_ref[...], (((1,), (1,)), ((), ())),
        preferred_element_type=jnp.float32)
    lt = lt + b_ref[...]
    m = jnp.max(lt, axis=0, keepdims=True)
    s = jnp.sum(jnp.exp(lt - m), axis=0, keepdims=True)
    v = 1.0 / s
    iota = jax.lax.broadcasted_iota(jnp.int32, (_NE, _TILE), 0)
    e = jnp.min(jnp.where(lt == m, iota, _NE), axis=0, keepdims=True)
    v_ref[0, 0, :] = v[0]
    e_ref[0, 0, :] = e[0]
    contrib = jnp.where(iota == e, v, 0.0)

    @pl.when(i == 0)
    def _():
        dacc_ref[...] = jnp.zeros_like(dacc_ref)

    dacc_ref[...] += contrib


def _pass2_body(v_ref, e_ref, dacc_ref, o_ref, *, capacity):
    denom = jnp.sum(dacc_ref[...], axis=1, keepdims=True) + _EPS
    recip = capacity / denom
    v = v_ref[0, 0, :][None, :]
    e = e_ref[0, 0, :][None, :]
    iota = jax.lax.broadcasted_iota(jnp.int32, (_NE, _TILE), 0)
    out_t = jnp.where(iota == e, v * recip, 0.0)
    o_ref[...] = out_t.T


def kernel(x, w_gate, b_gate):
    n, dim = x.shape
    ne = w_gate.shape[0]
    capacity = float(n)
    num_tiles = n // _TILE
    b2 = b_gate.reshape(ne, 1)

    v3, e3, dacc = pl.pallas_call(
        _pass1_body,
        grid=(num_tiles,),
        in_specs=[
            pl.BlockSpec((_TILE, dim), lambda i: (i, 0)),
            pl.BlockSpec((ne, dim), lambda i: (0, 0)),
            pl.BlockSpec((ne, 1), lambda i: (0, 0)),
        ],
        out_specs=[
            pl.BlockSpec((1, 1, _TILE), lambda i: (i, 0, 0)),
            pl.BlockSpec((1, 1, _TILE), lambda i: (i, 0, 0)),
            pl.BlockSpec((ne, _TILE), lambda i: (0, 0)),
        ],
        out_shape=[
            jax.ShapeDtypeStruct((num_tiles, 1, _TILE), jnp.float32),
            jax.ShapeDtypeStruct((num_tiles, 1, _TILE), jnp.int32),
            jax.ShapeDtypeStruct((ne, _TILE), jnp.float32),
        ],
        compiler_params=pltpu.CompilerParams(
            dimension_semantics=("arbitrary",)),
    )(x, w_gate, b2)

    out = pl.pallas_call(
        functools.partial(_pass2_body, capacity=capacity),
        grid=(num_tiles,),
        in_specs=[
            pl.BlockSpec((1, 1, _TILE), lambda i: (i, 0, 0)),
            pl.BlockSpec((1, 1, _TILE), lambda i: (i, 0, 0)),
            pl.BlockSpec((ne, _TILE), lambda i: (0, 0)),
        ],
        out_specs=pl.BlockSpec((_TILE, ne), lambda i: (i, 0)),
        out_shape=jax.ShapeDtypeStruct((n, ne), jnp.float32),
        compiler_params=pltpu.CompilerParams(
            dimension_semantics=("arbitrary",)),
    )(v3, e3, dacc)
    return out


# speedup vs baseline: 9.5273x; 1.1028x over previous
import functools

import jax
import jax.numpy as jnp
from jax.experimental import pallas as pl
from jax.experimental.pallas import tpu as pltpu

_NE = 64
_EPS = 1e-6
_TILE = 4096


def _pass1_body(x_ref, w_ref, b_ref, v_ref, e_ref, dacc_ref):
    i = pl.program_id(0)
    lt = jax.lax.dot_general(
        w_ref[...], x_ref[...], (((1,), (1,)), ((), ())),
        preferred_element_type=jnp.float32)
    lt = lt + b_ref[...]
    m = jnp.max(lt, axis=0, keepdims=True)
    s = jnp.sum(jnp.exp(lt - m), axis=0, keepdims=True)
    v = 1.0 / s
    iota = jax.lax.broadcasted_iota(jnp.int32, (_NE, _TILE), 0)
    e = jnp.min(jnp.where(lt == m, iota, _NE), axis=0, keepdims=True)
    v_ref[0, 0, :] = v[0]
    e_ref[0, 0, :] = e[0]
    contrib = jnp.where(iota == e, v, 0.0)

    @pl.when(i == 0)
    def _():
        dacc_ref[...] = jnp.zeros_like(dacc_ref)

    dacc_ref[...] += contrib


def _pass2_body(v_ref, e_ref, dacc_ref, o_ref, *, capacity):
    denom = jnp.sum(dacc_ref[...], axis=1, keepdims=True) + _EPS
    recip = capacity / denom
    v = v_ref[0, 0, :][None, :]
    e = e_ref[0, 0, :][None, :]
    iota = jax.lax.broadcasted_iota(jnp.int32, (_NE, _TILE), 0)
    out_t = jnp.where(iota == e, v * recip, 0.0)
    o_ref[...] = out_t.T


def kernel(x, w_gate, b_gate):
    n, dim = x.shape
    ne = w_gate.shape[0]
    capacity = float(n)
    num_tiles = n // _TILE
    b2 = b_gate.reshape(ne, 1)

    v3, e3, dacc = pl.pallas_call(
        _pass1_body,
        grid=(num_tiles,),
        in_specs=[
            pl.BlockSpec((_TILE, dim), lambda i: (i, 0)),
            pl.BlockSpec((ne, dim), lambda i: (0, 0)),
            pl.BlockSpec((ne, 1), lambda i: (0, 0)),
        ],
        out_specs=[
            pl.BlockSpec((1, 1, _TILE), lambda i: (i, 0, 0)),
            pl.BlockSpec((1, 1, _TILE), lambda i: (i, 0, 0)),
            pl.BlockSpec((ne, _TILE), lambda i: (0, 0)),
        ],
        out_shape=[
            jax.ShapeDtypeStruct((num_tiles, 1, _TILE), jnp.float32),
            jax.ShapeDtypeStruct((num_tiles, 1, _TILE), jnp.int32),
            jax.ShapeDtypeStruct((ne, _TILE), jnp.float32),
        ],
        compiler_params=pltpu.CompilerParams(
            dimension_semantics=("arbitrary",)),
    )(x, w_gate, b2)

    out = pl.pallas_call(
        functools.partial(_pass2_body, capacity=capacity),
        grid=(num_tiles,),
        in_specs=[
            pl.BlockSpec((1, 1, _TILE), lambda i: (i, 0, 0)),
            pl.BlockSpec((1, 1, _TILE), lambda i: (i, 0, 0)),
            pl.BlockSpec((ne, _TILE), lambda i: (0, 0)),
        ],
        out_specs=pl.BlockSpec((_TILE, ne), lambda i: (i, 0)),
        out_shape=jax.ShapeDtypeStruct((n, ne), jnp.float32),
        compiler_params=pltpu.CompilerParams(
            dimension_semantics=("arbitrary",)),
    )(v3, e3, dacc)
    return out
